# baseline (speedup 1.0000x reference)
import jax
import jax.numpy as jnp
from jax.experimental import pallas as pl

B, H, W = 16, 512, 512
N = B * H * W
ROWS = B * H
GRID = 16
BLK = ROWS // GRID


def _body(x_ref, t_ref, m_ref, sum_ref, cnt_ref):
    i = pl.program_id(0)

    @pl.when(i == 0)
    def _init():
        sum_ref[...] = jnp.zeros((1, 1), jnp.float32)
        cnt_ref[...] = jnp.zeros((1, 1), jnp.float32)

    d = x_ref[...] - t_ref[...]
    ad = jnp.abs(d)
    loss = jnp.where(ad < 1.0, 0.5 * d * d, ad - 0.5)
    m = m_ref[...].astype(jnp.float32)
    sum_ref[...] += jnp.sum(loss * m).reshape(1, 1)
    cnt_ref[...] += jnp.sum(m).reshape(1, 1)


def kernel(inputs, targets, mask):
    x = inputs.reshape(ROWS, W)
    t = targets.reshape(ROWS, W)
    m = mask.reshape(ROWS, W)
    s, c = pl.pallas_call(
        _body,
        grid=(GRID,),
        in_specs=[
            pl.BlockSpec((BLK, W), lambda i: (i, 0)),
            pl.BlockSpec((BLK, W), lambda i: (i, 0)),
            pl.BlockSpec((BLK, W), lambda i: (i, 0)),
        ],
        out_specs=[
            pl.BlockSpec((1, 1), lambda i: (0, 0)),
            pl.BlockSpec((1, 1), lambda i: (0, 0)),
        ],
        out_shape=[
            jax.ShapeDtypeStruct((1, 1), jnp.float32),
            jax.ShapeDtypeStruct((1, 1), jnp.float32),
        ],
    )(x, t, m)
    return s[0, 0] / jnp.maximum(c[0, 0], 1.0)
